# baseline (speedup 1.0000x reference)
import functools

import jax
import jax.numpy as jnp
from jax import lax
from jax.experimental import pallas as pl
from jax.experimental.pallas import tpu as pltpu
from jax.experimental.pallas import tpu_sc as plsc

N = 10000
E = 320000
D = 128

NC = 2
NS = 16
NW = NC * NS
EPW = E // NW
CHUNK = 80
SCHUNK = 25
SEDGE = SCHUNK * CHUNK
NSTAGE = EPW // SEDGE
RPW = 624
ZROWS = 16
TAIL = N - NS * RPW


def _spmm_sc(x, col, row4, val):
    mesh = plsc.VectorSubcoreMesh(core_axis_name="c", subcore_axis_name="s")

    @functools.partial(
        pl.kernel,
        out_type=jax.ShapeDtypeStruct((NC, N, D), jnp.float32),
        mesh=mesh,
        scratch_types=[
            pltpu.VMEM_SHARED((N, D), jnp.float32),
            pltpu.VMEM((SEDGE,), jnp.int32),
            pltpu.VMEM((SEDGE,), jnp.int32),
            pltpu.VMEM((SCHUNK, CHUNK), jnp.int32),
            pltpu.VMEM((SCHUNK, CHUNK), jnp.int32),
            pltpu.VMEM((SEDGE,), jnp.float32),
            pltpu.VMEM((SEDGE,), jnp.float32),
            pltpu.VMEM((CHUNK, D), jnp.float32),
            pltpu.VMEM((CHUNK, D), jnp.float32),
            pltpu.VMEM((CHUNK, D), jnp.float32),
            pltpu.VMEM((ZROWS, D), jnp.float32),
            pltpu.SemaphoreType.DMA,
            pltpu.SemaphoreType.DMA,
            pltpu.SemaphoreType.DMA,
            pltpu.SemaphoreType.DMA,
            pltpu.SemaphoreType.DMA,
            pltpu.SemaphoreType.DMA,
            pltpu.SemaphoreType.DMA,
        ],
    )
    def spmm(x_hbm, col_hbm, row_hbm, val_hbm, out_hbm,
             acc, col_a, col_b, row_a, row_b, val_a, val_b, g0, g1, g2, zbuf,
             sg0, sg1, sg2, ss0, ss1, ss2, sst):
        c = lax.axis_index("c")
        s = lax.axis_index("s")
        wid = s * NC + c
        cols = (col_a, col_b)
        rows = (row_a, row_b)
        vals = (val_a, val_b)

        def stage_copies(st_idx, p):
            base_e = wid * EPW + st_idx * SEDGE
            return (
                pltpu.make_async_copy(
                    col_hbm.at[pl.ds(base_e, SEDGE)], cols[p], sst),
                pltpu.make_async_copy(
                    row_hbm.at[wid, st_idx], rows[p], sst),
                pltpu.make_async_copy(
                    val_hbm.at[pl.ds(base_e, SEDGE)], vals[p], sst),
            )

        for cp in stage_copies(0, 0):
            cp.start()

        @pl.loop(0, ZROWS)
        def _zero(i):
            for t in range(D // 16):
                zbuf.at[i, pl.ds(t * 16, 16)][...] = jnp.zeros(
                    (16,), jnp.float32)

        def zero_copies():
            return [pltpu.make_async_copy(
                        zbuf, acc.at[pl.ds(s * RPW + j * ZROWS, ZROWS)], ss0)
                    for j in range(RPW // ZROWS)]

        for cp in zero_copies():
            cp.start()

        @pl.when(s == 0)
        def _zero_tail():
            pltpu.sync_copy(zbuf.at[pl.ds(0, TAIL)],
                            acc.at[pl.ds(NS * RPW, TAIL)])

        for cp in zero_copies():
            cp.wait()
        plsc.subcore_barrier()

        for st in range(NSTAGE):
            p = st % 2
            for cp in stage_copies(st, p):
                cp.wait()
            if st + 1 < NSTAGE:
                for cp in stage_copies(st + 1, 1 - p):
                    cp.start()

            gbufs = (g0, g1, g2)
            sgs = (sg0, sg1, sg2)
            sss = (ss0, ss1, ss2)

            def gather(k, r):
                cidx = cols[p].at[pl.ds(k * CHUNK, CHUNK)]
                return pltpu.make_async_copy(x_hbm.at[cidx], gbufs[r],
                                             sgs[r])

            def scatter(k, r):
                return pltpu.make_async_copy(gbufs[r], acc.at[rows[p].at[k]],
                                             sss[r])

            def scale(k, r):
                gb = gbufs[r]

                @pl.loop(0, CHUNK, step=16)
                def _scale(e0):
                    vals16 = vals[p][pl.ds(k * CHUNK + e0, 16)]
                    for j in range(16):
                        vv = lax.broadcast(vals16[j], (16,))
                        for t in range(D // 16):
                            sl = (e0 + j, pl.ds(t * 16, 16))
                            gb.at[sl][...] = gb.at[sl][...] * vv

            gather(0, 0).start()
            gather(1, 1).start()

            @pl.loop(0, SCHUNK // 3)
            def _triple(t):
                for r in range(3):
                    cc = 3 * t + r
                    if r == 0:
                        @pl.when(t > 0)
                        def _free():
                            scatter(cc - 1, 2).wait()

                        gather(cc + 2, 2).start()
                    elif r == 1:
                        scatter(cc - 1, 0).wait()
                        gather(cc + 2, 0).start()
                    else:
                        scatter(cc - 1, 1).wait()

                        @pl.when(t < SCHUNK // 3 - 1)
                        def _ahead():
                            gather(cc + 2, 1).start()

                    gather(cc, r).wait()
                    scale(cc, r)
                    scatter(cc, r).start(add=True)

            last = SCHUNK - 1
            scatter(last - 1, 2).wait()
            gather(last, 0).wait()
            scale(last, 0)
            scatter(last, 0).start(add=True)
            scatter(last, 0).wait()

        plsc.subcore_barrier()
        pltpu.sync_copy(acc.at[pl.ds(s * RPW, RPW)],
                        out_hbm.at[c, pl.ds(s * RPW, RPW)])

        @pl.when(s == 0)
        def _write_tail():
            pltpu.sync_copy(acc.at[pl.ds(NS * RPW, TAIL)],
                            out_hbm.at[c, pl.ds(NS * RPW, TAIL)])

    return spmm(x, col, row4, val)


def _linear_tc(parts, W, b, relu):
    def body(p_ref, w_ref, b_ref, o_ref):
        h = p_ref[0] + p_ref[1]
        y = jnp.dot(h, w_ref[...], preferred_element_type=jnp.float32)
        y = y + b_ref[...]
        if relu:
            y = jnp.maximum(y, 0.0)
        o_ref[...] = y

    blk = 2000
    return pl.pallas_call(
        body,
        grid=(N // blk,),
        in_specs=[pl.BlockSpec((NC, blk, D), lambda i: (0, i, 0)),
                  pl.BlockSpec((D, D), lambda i: (0, 0)),
                  pl.BlockSpec((1, D), lambda i: (0, 0))],
        out_specs=pl.BlockSpec((blk, D), lambda i: (i, 0)),
        out_shape=jax.ShapeDtypeStruct((N, D), jnp.float32),
    )(parts, W, b.reshape(1, D))


def kernel(x, edge_index, adj_values, W1, b1, W2, b2):
    row = edge_index[0]
    col = edge_index[1]
    row4 = row.reshape(NW, NSTAGE, SCHUNK, CHUNK)

    p1 = _spmm_sc(x, col, row4, adj_values)
    h = _linear_tc(p1, W1, b1, relu=True)
    p2 = _spmm_sc(h, col, row4, adj_values)
    out = _linear_tc(p2, W2, b2, relu=False)
    return out


# speedup vs baseline: 1.0419x; 1.0419x over previous
import functools

import jax
import jax.numpy as jnp
from jax import lax
from jax.experimental import pallas as pl
from jax.experimental.pallas import tpu as pltpu
from jax.experimental.pallas import tpu_sc as plsc

N = 10000
E = 320000
D = 128

NC = 2
NS = 16
NW = NC * NS
EPW = E // NW
CHUNK = 80
SCHUNK = 25
SEDGE = SCHUNK * CHUNK
NSTAGE = EPW // SEDGE
RPW = 624
ZROWS = 16
TAIL = N - NS * RPW


def _spmm_sc(x, col, row4, val):
    mesh = plsc.VectorSubcoreMesh(core_axis_name="c", subcore_axis_name="s")

    @functools.partial(
        pl.kernel,
        out_type=jax.ShapeDtypeStruct((NC, N, D), jnp.float32),
        mesh=mesh,
        scratch_types=[
            pltpu.VMEM_SHARED((N, D), jnp.float32),
            pltpu.VMEM((SEDGE,), jnp.int32),
            pltpu.VMEM((SEDGE,), jnp.int32),
            pltpu.VMEM((SCHUNK, CHUNK), jnp.int32),
            pltpu.VMEM((SCHUNK, CHUNK), jnp.int32),
            pltpu.VMEM((SEDGE,), jnp.float32),
            pltpu.VMEM((SEDGE,), jnp.float32),
            pltpu.VMEM((CHUNK, D), jnp.float32),
            pltpu.VMEM((CHUNK, D), jnp.float32),
            pltpu.VMEM((CHUNK, D), jnp.float32),
            pltpu.VMEM((ZROWS, D), jnp.float32),
            pltpu.SemaphoreType.DMA,
            pltpu.SemaphoreType.DMA,
            pltpu.SemaphoreType.DMA,
            pltpu.SemaphoreType.DMA,
            pltpu.SemaphoreType.DMA,
            pltpu.SemaphoreType.DMA,
            pltpu.SemaphoreType.DMA,
        ],
    )
    def spmm(x_hbm, col_hbm, row_hbm, val_hbm, out_hbm,
             acc, col_a, col_b, row_a, row_b, val_a, val_b, g0, g1, g2, zbuf,
             sg0, sg1, sg2, ss0, ss1, ss2, sst):
        c = lax.axis_index("c")
        s = lax.axis_index("s")
        wid = s * NC + c
        cols = (col_a, col_b)
        rows = (row_a, row_b)
        vals = (val_a, val_b)

        def stage_copies(st_idx, p):
            base_e = wid * EPW + st_idx * SEDGE
            return (
                pltpu.make_async_copy(
                    col_hbm.at[pl.ds(base_e, SEDGE)], cols[p], sst),
                pltpu.make_async_copy(
                    row_hbm.at[wid, st_idx], rows[p], sst),
                pltpu.make_async_copy(
                    val_hbm.at[pl.ds(base_e, SEDGE)], vals[p], sst),
            )

        for cp in stage_copies(0, 0):
            cp.start()

        @pl.loop(0, ZROWS)
        def _zero(i):
            for t in range(D // 16):
                zbuf.at[i, pl.ds(t * 16, 16)][...] = jnp.zeros(
                    (16,), jnp.float32)

        def zero_copies():
            return [pltpu.make_async_copy(
                        zbuf, acc.at[pl.ds(s * RPW + j * ZROWS, ZROWS)], ss0)
                    for j in range(RPW // ZROWS)]

        for cp in zero_copies():
            cp.start()

        @pl.when(s == 0)
        def _zero_tail():
            pltpu.sync_copy(zbuf.at[pl.ds(0, TAIL)],
                            acc.at[pl.ds(NS * RPW, TAIL)])

        for cp in zero_copies():
            cp.wait()
        plsc.subcore_barrier()

        gbufs = (g0, g1, g2)
        sgs = (sg0, sg1, sg2)
        sss = (ss0, ss1, ss2)

        def gather(pp, k, r):
            cidx = cols[pp].at[pl.ds(k * CHUNK, CHUNK)]
            return pltpu.make_async_copy(x_hbm.at[cidx], gbufs[r], sgs[r])

        def scatter(pp, k, r):
            return pltpu.make_async_copy(gbufs[r], acc.at[rows[pp].at[k]],
                                         sss[r])

        def scale(pp, k, r):
            gb = gbufs[r]

            @pl.loop(0, CHUNK, step=16)
            def _scale(e0):
                vals16 = vals[pp][pl.ds(k * CHUNK + e0, 16)]
                for j in range(16):
                    vv = lax.broadcast(vals16[j], (16,))
                    for tt in range(D // 16):
                        sl = (e0 + j, pl.ds(tt * 16, 16))
                        gb.at[sl][...] = gb.at[sl][...] * vv

        for cp in stage_copies(0, 0):
            cp.wait()
        gather(0, 0, 0).start()
        gather(0, 1, 1).start()

        for st in range(NSTAGE):
            p = st % 2
            q = 1 - p
            if st + 1 < NSTAGE:
                for cp in stage_copies(st + 1, q):
                    cp.start()

            @pl.loop(0, SCHUNK // 6)
            def _six(t):
                for r in range(6):
                    cc = 6 * t + r
                    slot = (st + r) % 3
                    fslot = (st + r + 2) % 3

                    if r == 0:
                        @pl.when(t > 0)
                        def _a():
                            scatter(p, cc - 1, fslot).wait()

                        if st > 0:
                            @pl.when(t == 0)
                            def _a2():
                                scatter(q, SCHUNK - 1, fslot).wait()
                    else:
                        scatter(p, cc - 1, fslot).wait()

                    if r < 5:
                        gather(p, cc + 2, fslot).start()
                    else:
                        @pl.when(t < SCHUNK // 6 - 1)
                        def _b():
                            gather(p, cc + 2, fslot).start()

                        if st + 1 < NSTAGE:
                            @pl.when(t == SCHUNK // 6 - 1)
                            def _b2():
                                for cp in stage_copies(st + 1, q):
                                    cp.wait()
                                gather(q, 0, fslot).start()

                    gather(p, cc, slot).wait()
                    scale(p, cc, slot)
                    scatter(p, cc, slot).start(add=True)

            sl24 = st % 3
            scatter(p, SCHUNK - 2, (st + 2) % 3).wait()
            if st + 1 < NSTAGE:
                gather(q, 1, (st + 2) % 3).start()
            gather(p, SCHUNK - 1, sl24).wait()
            scale(p, SCHUNK - 1, sl24)
            scatter(p, SCHUNK - 1, sl24).start(add=True)
            if st + 1 == NSTAGE:
                scatter(p, SCHUNK - 1, sl24).wait()

        plsc.subcore_barrier()
        pltpu.sync_copy(acc.at[pl.ds(s * RPW, RPW)],
                        out_hbm.at[c, pl.ds(s * RPW, RPW)])

        @pl.when(s == 0)
        def _write_tail():
            pltpu.sync_copy(acc.at[pl.ds(NS * RPW, TAIL)],
                            out_hbm.at[c, pl.ds(NS * RPW, TAIL)])

    return spmm(x, col, row4, val)


def _linear_tc(parts, W, b, relu):
    def body(p_ref, w_ref, b_ref, o_ref):
        h = p_ref[0] + p_ref[1]
        y = jnp.dot(h, w_ref[...], preferred_element_type=jnp.float32)
        y = y + b_ref[...]
        if relu:
            y = jnp.maximum(y, 0.0)
        o_ref[...] = y

    blk = 2000
    return pl.pallas_call(
        body,
        grid=(N // blk,),
        in_specs=[pl.BlockSpec((NC, blk, D), lambda i: (0, i, 0)),
                  pl.BlockSpec((D, D), lambda i: (0, 0)),
                  pl.BlockSpec((1, D), lambda i: (0, 0))],
        out_specs=pl.BlockSpec((blk, D), lambda i: (i, 0)),
        out_shape=jax.ShapeDtypeStruct((N, D), jnp.float32),
    )(parts, W, b.reshape(1, D))


def kernel(x, edge_index, adj_values, W1, b1, W2, b2):
    row = edge_index[0]
    col = edge_index[1]
    row4 = row.reshape(NW, NSTAGE, SCHUNK, CHUNK)

    p1 = _spmm_sc(x, col, row4, adj_values)
    h = _linear_tc(p1, W1, b1, relu=True)
    p2 = _spmm_sc(h, col, row4, adj_values)
    out = _linear_tc(p2, W2, b2, relu=False)
    return out
